# Initial kernel scaffold; baseline (speedup 1.0000x reference)
import jax
import jax.numpy as jnp
from jax.experimental import pallas as pl


def kernel(preds, low_feats, high_feats, unlabeled_ROIs):
    raise NotImplementedError("write your pallas kernel here")


# speedup vs baseline: 89.4881x; 89.4881x over previous
import functools
import numpy as np
import jax
import jax.numpy as jnp
from jax import lax
from jax.experimental import pallas as pl
from jax.experimental.pallas import tpu as pltpu
from jax.experimental.pallas import tpu_sc as plsc

H = 64
W = 64
N = H * W
EH = H * (W - 1)
E = 2 * EH
C = 21
NCH = C + 1
SIGMA_LOW = 0.02
SIGMA_HIGH = 1.0

ORD = 0
UF = 8064
ADJN = 12160
DEG = 28544
QUE = 32640
PAR = 36736
IBUF_WORDS = 40832
PARB = 0
QUEB = 4096
WE = 0
WN = 8064
ADJW = 12160
ABUF = 0
WX = 65536
DEN = 69632
FBUF_WORDS = 73728


def _tree_sum(get, n):
    if n < 8:
        r = get(0)
        for i in range(1, n):
            r = r + get(i)
        return r
    if n <= 128:
        r = [get(j) for j in range(8)]
        nb = n - (n % 8)
        for i in range(8, nb, 8):
            for j in range(8):
                r[j] = r[j] + get(i + j)
        res = ((r[0] + r[1]) + (r[2] + r[3])) + ((r[4] + r[5]) + (r[6] + r[7]))
        for i in range(nb, n):
            res = res + get(i)
        return res
    n2 = n // 2
    n2 -= n2 % 8
    return _tree_sum(get, n2) + _tree_sum(lambda i: get(n2 + i), n - n2)


def _edge_weight_kernel(f_ref, wh_ref, wv_ref):
    x = f_ref[...]
    dh = x[:, :, :-1] - x[:, :, 1:]
    dv = x[:, :-1, :] - x[:, 1:, :]
    sh = dh * dh
    sv = dv * dv
    c = x.shape[0]
    wh_ref[...] = _tree_sum(lambda i: sh[i], c)
    wv_ref[...] = _tree_sum(lambda i: sv[i], c)


def _edge_weights(feat):
    c = feat.shape[0]
    wh, wv = pl.pallas_call(
        _edge_weight_kernel,
        out_shape=(
            jax.ShapeDtypeStruct((H, W - 1), jnp.float32),
            jax.ShapeDtypeStruct((H - 1, W), jnp.float32),
        ),
    )(feat)
    return jnp.concatenate([wh.reshape(-1), wv.reshape(-1)], axis=0)


def _sigmoid_kernel(x_ref, o_ref):
    o_ref[...] = jax.nn.sigmoid(x_ref[...])


def _loss_kernel(p_ref, a_ref, r_ref, loss_ref, n_ref):
    acc = jnp.float32(0.0)
    for b in range(2):
        d = jnp.abs(p_ref[b * C:(b + 1) * C, :] - a_ref[b * C:(b + 1) * C, :])
        acc = acc + jnp.sum(r_ref[b:b + 1, :] * d)
    loss_ref[...] = acc.reshape(1, 1)
    n_ref[...] = jnp.sum(r_ref[...]).reshape(1, 1)


def _iota16():
    return lax.broadcasted_iota(jnp.int32, (16,), 0)


def _sc_body(order_hbm, we_hbm, x_hbm, out_hbm,
             ibuf, fbuf, parent_sh, queue_sh, wexp_sh, denom_sh):
    cid = lax.axis_index("c")
    sid = lax.axis_index("s")
    iota = _iota16()
    mask0 = iota == 0

    def splat(x):
        return jnp.full((16,), x, dtype=jnp.int32)

    def gi(idx):
        return plsc.load_gather(ibuf, [idx])

    def gf(idx):
        return plsc.load_gather(fbuf, [idx])

    def si(idx, val, mask=None):
        plsc.store_scatter(ibuf, [idx], val, mask=mask)

    def sf(idx, val, mask=None):
        plsc.store_scatter(fbuf, [idx], val, mask=mask)

    @pl.when(sid < 2)
    def _phase_a():
        tree = cid * 2 + sid
        pltpu.sync_copy(order_hbm.at[tree], ibuf.at[pl.ds(ORD, E)])
        pltpu.sync_copy(we_hbm.at[tree], fbuf.at[pl.ds(WE, E)])

        def init_body(i, _):
            base = splat(i * 16) + iota
            si(splat(UF) + base, base)
            si(splat(DEG) + base, jnp.zeros((16,), jnp.int32))
            si(splat(PAR) + base, jnp.full((16,), -1, jnp.int32))
            return 0

        lax.fori_loop(0, N // 16, init_body, 0)
        si(splat(PAR), jnp.zeros((16,), jnp.int32), mask=mask0)
        si(splat(QUE), jnp.zeros((16,), jnp.int32), mask=mask0)
        sf(splat(WN), jnp.full((16,), 1e30, jnp.float32), mask=mask0)

        def find(x):
            def cond(cur):
                return jnp.any(gi(splat(UF) + cur) != cur)

            def body(cur):
                p = gi(splat(UF) + cur)
                gp = gi(splat(UF) + p)
                si(splat(UF) + cur, gp, mask=mask0)
                return gp

            return lax.while_loop(cond, body, x)

        def edge_body(e, _):
            eid = gi(splat(ORD + e))
            is_h = eid < EH
            r = eid // (W - 1)
            cc = eid - r * (W - 1)
            k = eid - EH
            u = jnp.where(is_h, r * W + cc, k)
            v = jnp.where(is_h, r * W + cc + 1, k + W)
            ru = find(u)
            rv = find(v)
            take = jnp.any(ru != rv)

            @pl.when(take)
            def _():
                si(splat(UF) + ru, rv, mask=mask0)
                wev = gf(splat(WE) + eid)
                du = gi(splat(DEG) + u)
                si(splat(ADJN) + u * 4 + du, v, mask=mask0)
                sf(splat(ADJW) + u * 4 + du, wev, mask=mask0)
                si(splat(DEG) + u, du + 1, mask=mask0)
                dv = gi(splat(DEG) + v)
                si(splat(ADJN) + v * 4 + dv, u, mask=mask0)
                sf(splat(ADJW) + v * 4 + dv, wev, mask=mask0)
                si(splat(DEG) + v, dv + 1, mask=mask0)

            return 0

        lax.fori_loop(0, E, edge_body, 0)

        def bfs_body(t, tail):
            node = gi(splat(QUE + t))
            degn = gi(splat(DEG) + node)
            m = iota < degn
            slot_idx = jnp.where(m, node * 4 + iota, 0)
            nbr = gi(splat(ADJN) + slot_idx)
            wv = gf(splat(ADJW) + slot_idx)
            pv = gi(splat(PAR) + jnp.where(m, nbr, 0))
            new = m & (pv < 0)
            cnt = plsc.cumsum(new.astype(jnp.int32))
            pos = jnp.where(new, tail + cnt - 1, 0)
            si(splat(QUE) + pos, nbr, mask=new)
            si(splat(PAR) + jnp.where(new, nbr, 0), node, mask=new)
            sf(splat(WN) + jnp.where(new, nbr, 0), wv, mask=new)
            return tail + plsc.all_reduce_population_count(new)

        lax.fori_loop(0, N, bfs_body, splat(1))

        sigma = jnp.where(sid == 0, jnp.float32(SIGMA_LOW),
                          jnp.float32(SIGMA_HIGH))

        def exp_body(i, _):
            idx = splat(WN + i * 16) + iota
            sf(idx, jnp.exp(-(gf(idx) / sigma)))
            return 0

        lax.fori_loop(0, N // 16, exp_body, 0)

        pltpu.sync_copy(ibuf.at[pl.ds(PAR, N)], parent_sh.at[sid])
        pltpu.sync_copy(ibuf.at[pl.ds(QUE, N)], queue_sh.at[sid])
        pltpu.sync_copy(fbuf.at[pl.ds(WN, N)], wexp_sh.at[sid])

    plsc.subcore_barrier()

    @pl.when(sid < 2)
    def _load_x():
        pltpu.sync_copy(x_hbm.at[cid, sid], fbuf.at[pl.ds(ABUF, N * 16)])

    for lvl in range(2):
        @pl.when(sid < 2)
        def _sweeps(lvl=lvl):
            pltpu.sync_copy(parent_sh.at[lvl], ibuf.at[pl.ds(PARB, N)])
            pltpu.sync_copy(queue_sh.at[lvl], ibuf.at[pl.ds(QUEB, N)])
            pltpu.sync_copy(wexp_sh.at[lvl], fbuf.at[pl.ds(WX, N)])

            def down_body(t, _):
                node = gi(splat(QUEB + N - 1 - t))
                p = gi(splat(PARB) + node)
                wv = gf(splat(WX) + node)
                arow = gf(splat(ABUF) + node * 16 + iota)
                plsc.addupdate_scatter(fbuf, [splat(ABUF) + p * 16 + iota],
                                       wv * arow)
                return 0

            lax.fori_loop(0, N, down_body, 0)

            def up_body(t, _):
                node = gi(splat(QUEB + t))
                p = gi(splat(PARB) + node)
                wv = gf(splat(WX) + node)
                arow = gf(splat(ABUF) + node * 16 + iota)
                fp = gf(splat(ABUF) + p * 16 + iota)
                sf(splat(ABUF) + node * 16 + iota,
                   arow + wv * (fp - wv * arow))
                return 0

            lax.fori_loop(0, N, up_body, 0)

            @pl.when(sid == 1)
            def _push_denom():
                def den_body(i, _):
                    base = splat(i * 16) + iota
                    sf(splat(DEN) + base, gf(splat(ABUF) + base * 16 + 5))
                    return 0

                lax.fori_loop(0, N // 16, den_body, 0)
                pltpu.sync_copy(fbuf.at[pl.ds(DEN, N)], denom_sh)

        plsc.subcore_barrier()

        @pl.when(sid < 2)
        def _normalize(lvl=lvl):
            pltpu.sync_copy(denom_sh, fbuf.at[pl.ds(DEN, N)])

            def norm_body(t, _):
                d = jnp.maximum(gf(splat(DEN + t)), jnp.float32(1e-12))
                idx = splat(ABUF + t * 16) + iota
                v = gf(idx) / d
                v = jnp.where(sid == 1,
                              jnp.where(iota == 5, jnp.float32(1.0),
                                        jnp.where(iota < 5, v,
                                                  jnp.float32(0.0))),
                              v)
                sf(idx, v)
                return 0

            lax.fori_loop(0, N, norm_body, 0)

        plsc.subcore_barrier()

    @pl.when(sid < 2)
    def _store_out():
        pltpu.sync_copy(fbuf.at[pl.ds(ABUF, N * 16)], out_hbm.at[cid, sid])


@jax.jit
def kernel(preds, low_feats, high_feats, unlabeled_ROIs):
    B = preds.shape[0]

    prob = pl.pallas_call(
        _sigmoid_kernel,
        out_shape=jax.ShapeDtypeStruct(preds.shape, jnp.float32),
    )(preds)

    w_all = jnp.stack([_edge_weights(low_feats[0]),
                       _edge_weights(high_feats[0]),
                       _edge_weights(low_feats[1]),
                       _edge_weights(high_feats[1])], axis=0)
    order_all = jnp.argsort(w_all, axis=1, stable=True).astype(jnp.int32)

    xt = prob.reshape(B, C, N).transpose(0, 2, 1)
    pad = jnp.zeros((B, N, 11), jnp.float32)
    ones = jnp.ones((B, N, 1), jnp.float32)
    xp = jnp.concatenate([xt, ones, pad[..., :10]], axis=2)
    x_in = xp.reshape(B, N, 2, 16).transpose(0, 2, 1, 3).reshape(B, 2, N * 16)

    sc = functools.partial(
        pl.kernel,
        out_type=jax.ShapeDtypeStruct((B, 2, N * 16), jnp.float32),
        mesh=plsc.VectorSubcoreMesh(core_axis_name="c", subcore_axis_name="s",
                                    num_cores=2, num_subcores=16),
        scratch_types=[
            pltpu.VMEM((IBUF_WORDS,), jnp.int32),
            pltpu.VMEM((FBUF_WORDS,), jnp.float32),
            pltpu.VMEM_SHARED((2, N), jnp.int32),
            pltpu.VMEM_SHARED((2, N), jnp.int32),
            pltpu.VMEM_SHARED((2, N), jnp.float32),
            pltpu.VMEM_SHARED((N,), jnp.float32),
        ],
        compiler_params=pltpu.CompilerParams(needs_layout_passes=False),
    )
    out = sc(_sc_body)(order_all, w_all, x_in)

    AS = (out.reshape(B, 2, N, 16).transpose(0, 2, 1, 3)
          .reshape(B, N, 32)[..., :C])
    AS = AS.transpose(0, 2, 1).reshape(B * C, N)

    ri = (jnp.arange(H) * (unlabeled_ROIs.shape[2] // H)).astype(jnp.int32)
    ci = (jnp.arange(W) * (unlabeled_ROIs.shape[3] // W)).astype(jnp.int32)
    roi = unlabeled_ROIs[:, 0][:, ri, :][:, :, ci].reshape(B, N)

    loss, n = pl.pallas_call(
        _loss_kernel,
        out_shape=(
            jax.ShapeDtypeStruct((1, 1), jnp.float32),
            jax.ShapeDtypeStruct((1, 1), jnp.float32),
        ),
    )(prob.reshape(B * C, N), AS, roi)
    loss = loss[0, 0]
    n = n[0, 0]
    return jnp.where(n > 0, loss / jnp.maximum(n, 1.0), jnp.float32(0.0))
